# baseline (speedup 1.0000x reference)
import dataclasses
import functools

import jax
import jax.numpy as jnp
from jax import lax
from jax.experimental import pallas as pl
from jax.experimental.pallas import tpu as pltpu
from jax.experimental.pallas import tpu_sc as plsc

NC, NS, L = 2, 16, 16
NW = NC * NS
CHUNK = 128


def _a_body(x_ref, w1_ref, enc_ref, w2_ref, dinv_ref, g_ref):
    c = jnp.sum(enc_ref[...] * w2_ref[...], axis=0, keepdims=True)
    ht = (
        jnp.dot(x_ref[...], w1_ref[...], preferred_element_type=jnp.float32)
        + c
    )
    g_ref[...] = ht * dinv_ref[...]


def _e_body(p_ref, g_ref, dinv_ref, bg_ref, gam_ref, bet_ref, wl_ref,
            bl_ref, out_ref):
    n = out_ref.shape[0]
    agg = p_ref[0, :n, :] + p_ref[1, :n, :] + g_ref[...]
    out0 = agg * dinv_ref[...] + bg_ref[...]
    mean = jnp.mean(out0, axis=0, keepdims=True)
    d = out0 - mean
    var = jnp.mean(d * d, axis=0, keepdims=True)
    y = d * lax.rsqrt(var + 1e-5) * gam_ref[...] + bet_ref[...]
    y = jnp.where(y >= 0, y, 0.01 * y)
    out_ref[...] = (
        jnp.dot(y, wl_ref[...], preferred_element_type=jnp.float32)
        + bl_ref[...]
    )


def _deg_dinv_body(n_pad, n_chunks, col_hbm, w_hbm, dinv_hbm,
                   colv, wv, dbuf, acc, sem):
    c = lax.axis_index("core")
    s = lax.axis_index("subcore")
    per_tile = n_chunks // NS
    base_chunk = s * per_tile
    rows_z = n_pad // NS
    pltpu.async_copy(col_hbm.at[pl.ds(base_chunk, per_tile)], colv, sem).wait()
    pltpu.async_copy(w_hbm.at[pl.ds(base_chunk, per_tile)], wv, sem).wait()

    @pl.loop(0, rows_z, step=L)
    def _(i):
        dbuf[pl.ds(i, L)] = jnp.zeros((L,), jnp.float32)

    pltpu.sync_copy(dbuf, acc.at[pl.ds(s * rows_z, rows_z)])
    plsc.subcore_barrier()

    @pl.loop(0, per_tile)
    def _(k):
        pltpu.async_copy(wv.at[k], acc.at[colv.at[k]], sem, add=True)

    @pl.loop(0, per_tile)
    def _(k):
        pltpu.make_async_copy(wv.at[k], acc.at[colv.at[k]], sem).wait()

    plsc.subcore_barrier()

    rows_d = n_pad // NW
    wid = c * NS + s
    base = wid * rows_d
    pltpu.sync_copy(acc.at[pl.ds(base, rows_d)], dbuf.at[pl.ds(0, rows_d)])

    @pl.loop(0, rows_d, step=L)
    def _(i):
        deg = dbuf[pl.ds(i, L)] + 1.0
        ii = plsc.bitcast(deg, jnp.int32)
        y = plsc.bitcast(jnp.int32(0x5F3759DF) - (ii >> 1), jnp.float32)
        for _ in range(3):
            y = y * (1.5 - 0.5 * deg * y * y)
        dbuf[pl.ds(i, L)] = y

    pltpu.sync_copy(dbuf.at[pl.ds(0, rows_d)], dinv_hbm.at[pl.ds(base, rows_d)])


NBUF = 4
CHD = 80


def _agg_body(n_pad, n_chunks, g_hbm, rw_hbm, col_hbm, out_hbm,
              ibuf, colb, buf, sbuf, acc, gsem, ssem):
    c = lax.axis_index("core")
    s = lax.axis_index("subcore")
    wid = c * NS + s
    per_tile = n_chunks // NW
    base_chunk = wid * per_tile

    @pl.loop(0, CHD)
    def _(r):
        for j in range(8):
            sbuf[0, r, pl.ds(j * L, L)] = jnp.zeros((L,), jnp.float32)

    rows_z = n_pad // NS
    for z in range(rows_z // CHD):
        pltpu.sync_copy(sbuf.at[0], acc.at[pl.ds(s * rows_z + z * CHD, CHD)])
    plsc.subcore_barrier()

    rww = 128 + CHD

    def fetch_idx(k, h):
        pltpu.async_copy(
            rw_hbm.at[pl.ds((base_chunk + k) * rww, rww)],
            ibuf.at[h, 0], gsem.at[h])
        pltpu.async_copy(col_hbm.at[pl.ds((base_chunk + k) * CHD, CHD)],
                         colb.at[h, 0], gsem.at[h])

    def wait_idx(k, h):
        pltpu.make_async_copy(
            rw_hbm.at[pl.ds((base_chunk + k) * rww, rww)],
            ibuf.at[h, 0], gsem.at[h]).wait()
        pltpu.make_async_copy(col_hbm.at[pl.ds((base_chunk + k) * CHD, CHD)],
                              colb.at[h, 0], gsem.at[h]).wait()

    def row_idx(h):
        return ibuf.at[h, 0, pl.ds(128, CHD)]

    def scale(h, hs):
        @pl.loop(0, CHD)
        def _(e):
            wi = plsc.load_gather(ibuf.at[h, 0],
                                  [jnp.zeros((L,), jnp.int32) + e])
            wsp = plsc.bitcast(wi, jnp.float32)
            for j in range(4):
                v = buf[h, e, pl.ds(j * L, L)]
                lo = plsc.bitcast(v << 16, jnp.float32)
                hi = plsc.bitcast(v & jnp.int32(-65536), jnp.float32)
                sbuf[hs, e, pl.ds(j * 2 * L, L)] = lo * wsp
                sbuf[hs, e, pl.ds(j * 2 * L + L, L)] = hi * wsp

    fetch_idx(0, 0)
    fetch_idx(1, 1)
    wait_idx(0, 0)
    pltpu.async_copy(g_hbm.at[row_idx(0)], buf.at[0], gsem.at[0])

    @pl.loop(0, per_tile, step=NBUF)
    def _(k0):
        for h in range(NBUF):
            k = k0 + h
            hn = (h + 1) % NBUF
            hg = (h + 2) % NBUF

            hs = h % 2

            @pl.when(k + 1 < per_tile)
            def _():
                wait_idx(k + 1, hn)
                pltpu.async_copy(g_hbm.at[row_idx(hn)], buf.at[hn],
                                 gsem.at[hn])

            @pl.when(k >= 2)
            def _():
                pltpu.make_async_copy(
                    sbuf.at[hs], acc.at[colb.at[hg, 0]], ssem.at[hs]).wait()

            @pl.when(k + 2 < per_tile)
            def _():
                fetch_idx(k + 2, hg)

            pltpu.make_async_copy(g_hbm.at[row_idx(h)], buf.at[h],
                                  gsem.at[h]).wait()
            scale(h, hs)
            pltpu.async_copy(sbuf.at[hs], acc.at[colb.at[h, 0]], ssem.at[hs],
                             add=True)

    for h in (NBUF - 2, NBUF - 1):
        pltpu.make_async_copy(
            sbuf.at[h % 2], acc.at[colb.at[h, 0]], ssem.at[h % 2]).wait()

    plsc.subcore_barrier()
    pltpu.sync_copy(acc.at[pl.ds(s * rows_z, rows_z)],
                    out_hbm.at[c, pl.ds(s * rows_z, rows_z)])


def kernel(x, edge_index, edge_weight, time_diff, is_weekend,
           workday_freq, weekday_freq, W_gcn, b_gcn,
           bn_gamma, bn_beta, W_lin, b_lin):
    n, c_in = x.shape
    c_out = W_gcn.shape[1]
    t_feat = W_gcn.shape[0] - c_in
    e = edge_index.shape[1]

    n_pad = ((n + NW * L - 1) // (NW * L)) * (NW * L)
    align = 81920
    e_pad = ((e + align - 1) // align) * align
    n_chunks = e_pad // CHUNK
    pad = e_pad - e
    pad_idx = jnp.arange(pad, dtype=jnp.int32) % n
    row_p = jnp.concatenate([edge_index[0], pad_idx])
    col_p = jnp.concatenate([edge_index[1], pad_idx])
    w_p = jnp.concatenate([edge_weight, jnp.zeros((pad,), jnp.float32)])
    col2d = col_p.reshape(n_chunks, CHUNK)
    w2d = w_p.reshape(n_chunks, CHUNK)

    enc = jnp.where(
        jnp.asarray(is_weekend),
        jnp.sin(time_diff * workday_freq * jnp.pi),
        jnp.cos(time_diff * weekday_freq * jnp.pi),
    )
    enc_col = enc.reshape(t_feat, 1)
    W1 = W_gcn[:c_in]
    W2 = W_gcn[c_in:]

    mesh = plsc.VectorSubcoreMesh(core_axis_name="core",
                                  subcore_axis_name="subcore",
                                  num_cores=NC, num_subcores=NS)
    sc_params = pltpu.CompilerParams()
    if "needs_layout_passes" in pltpu.CompilerParams.__dataclass_fields__:
        sc_params = dataclasses.replace(sc_params, needs_layout_passes=False)
    per_tile_b = n_chunks // NS
    dinv_pad = pl.kernel(
        functools.partial(_deg_dinv_body, n_pad, n_chunks),
        out_type=jax.ShapeDtypeStruct((n_pad,), jnp.float32),
        mesh=mesh,
        scratch_types=[
            pltpu.VMEM((per_tile_b, CHUNK), jnp.int32),
            pltpu.VMEM((per_tile_b, CHUNK), jnp.float32),
            pltpu.VMEM((n_pad // NS,), jnp.float32),
            pltpu.VMEM_SHARED((n_pad,), jnp.float32),
            pltpu.SemaphoreType.DMA,
        ],
        compiler_params=sc_params,
    )(col2d, w2d)
    dinv_col = dinv_pad[:n].reshape(n, 1)

    g = pl.pallas_call(
        _a_body,
        out_shape=jax.ShapeDtypeStruct((n, c_out), jnp.float32),
    )(x, W1, enc_col, W2, dinv_col)

    nchd = e_pad // CHD
    b16 = lax.bitcast_convert_type(g.astype(jnp.bfloat16), jnp.uint16)
    b16 = b16.astype(jnp.uint32).reshape(n, 4, 2, L)
    gbi = lax.bitcast_convert_type(
        b16[:, :, 0, :] | (b16[:, :, 1, :] << 16),
        jnp.int32).reshape(n, c_out // 2)
    w_bits = lax.bitcast_convert_type(w_p, jnp.int32)
    rw = jnp.concatenate(
        [w_bits.reshape(nchd, CHD),
         jnp.zeros((nchd, 128 - CHD), jnp.int32),
         row_p.reshape(nchd, CHD)], axis=1).reshape(-1)
    partials = pl.kernel(
        functools.partial(_agg_body, n_pad, nchd),
        out_type=jax.ShapeDtypeStruct((NC, n_pad, c_out), jnp.float32),
        mesh=mesh,
        scratch_types=[
            pltpu.VMEM((NBUF, 1, 128 + CHD), jnp.int32),
            pltpu.VMEM((NBUF, 1, CHD), jnp.int32),
            pltpu.VMEM((NBUF, CHD, c_out // 2), jnp.int32),
            pltpu.VMEM((2, CHD, c_out), jnp.float32),
            pltpu.VMEM_SHARED((n_pad, c_out), jnp.float32),
            pltpu.SemaphoreType.DMA((NBUF,)),
            pltpu.SemaphoreType.DMA((2,)),
        ],
        compiler_params=dataclasses.replace(sc_params,
                                            use_tc_tiling_on_sc=False),
    )(gbi, rw, col_p)

    out = pl.pallas_call(
        _e_body,
        out_shape=jax.ShapeDtypeStruct((n, c_out), jnp.float32),
    )(partials, g, dinv_col, b_gcn.reshape(1, c_out),
      bn_gamma.reshape(1, c_out), bn_beta.reshape(1, c_out),
      W_lin, b_lin.reshape(1, c_out))
    return out


# speedup vs baseline: 1.6074x; 1.6074x over previous
import dataclasses
import functools

import jax
import jax.numpy as jnp
from jax import lax
from jax.experimental import pallas as pl
from jax.experimental.pallas import tpu as pltpu
from jax.experimental.pallas import tpu_sc as plsc

NC, NS, L = 2, 16, 16
NW = NC * NS
CHUNK = 128


def _a_body(x_ref, w1_ref, enc_ref, w2_ref, dinv_ref, g_ref):
    c = jnp.sum(enc_ref[...] * w2_ref[...], axis=0, keepdims=True)
    ht = (
        jnp.dot(x_ref[...], w1_ref[...], preferred_element_type=jnp.float32)
        + c
    )
    g_ref[...] = ht * dinv_ref[...]


def _e_body(p_ref, g_ref, dinv_ref, bg_ref, gam_ref, bet_ref, wl_ref,
            bl_ref, out_ref):
    n = out_ref.shape[0]
    agg = p_ref[0, :n, :] + p_ref[1, :n, :] + g_ref[...]
    out0 = agg * dinv_ref[...] + bg_ref[...]
    mean = jnp.mean(out0, axis=0, keepdims=True)
    d = out0 - mean
    var = jnp.mean(d * d, axis=0, keepdims=True)
    y = d * lax.rsqrt(var + 1e-5) * gam_ref[...] + bet_ref[...]
    y = jnp.where(y >= 0, y, 0.01 * y)
    out_ref[...] = (
        jnp.dot(y, wl_ref[...], preferred_element_type=jnp.float32)
        + bl_ref[...]
    )


def _deg_dinv_body(n_pad, n_chunks, col_hbm, w_hbm, dinv_hbm,
                   colv, wv, dbuf, acc, sem):
    c = lax.axis_index("core")
    s = lax.axis_index("subcore")
    per_tile = n_chunks // NS
    base_chunk = s * per_tile
    rows_z = n_pad // NS
    pltpu.async_copy(col_hbm.at[pl.ds(base_chunk, per_tile)], colv, sem).wait()
    pltpu.async_copy(w_hbm.at[pl.ds(base_chunk, per_tile)], wv, sem).wait()

    @pl.loop(0, rows_z, step=L)
    def _(i):
        dbuf[pl.ds(i, L)] = jnp.zeros((L,), jnp.float32)

    pltpu.sync_copy(dbuf, acc.at[pl.ds(s * rows_z, rows_z)])
    plsc.subcore_barrier()

    @pl.loop(0, per_tile)
    def _(k):
        pltpu.async_copy(wv.at[k], acc.at[colv.at[k]], sem, add=True)

    @pl.loop(0, per_tile)
    def _(k):
        pltpu.make_async_copy(wv.at[k], acc.at[colv.at[k]], sem).wait()

    plsc.subcore_barrier()

    rows_d = n_pad // NW
    wid = c * NS + s
    base = wid * rows_d
    pltpu.sync_copy(acc.at[pl.ds(base, rows_d)], dbuf.at[pl.ds(0, rows_d)])

    @pl.loop(0, rows_d, step=L)
    def _(i):
        deg = dbuf[pl.ds(i, L)] + 1.0
        ii = plsc.bitcast(deg, jnp.int32)
        y = plsc.bitcast(jnp.int32(0x5F3759DF) - (ii >> 1), jnp.float32)
        for _ in range(3):
            y = y * (1.5 - 0.5 * deg * y * y)
        dbuf[pl.ds(i, L)] = y

    pltpu.sync_copy(dbuf.at[pl.ds(0, rows_d)], dinv_hbm.at[pl.ds(base, rows_d)])


NBUF = 4
CHD = 80


def _agg_body(n_pad, n_chunks, g_hbm, row_hbm, col_hbm, w_hbm, out_hbm,
              ibuf, buf, acc, gsem, ssem):
    c = lax.axis_index("core")
    s = lax.axis_index("subcore")
    wid = c * NS + s
    per_tile = n_chunks // NW
    base_chunk = wid * per_tile

    @pl.loop(0, CHD)
    def _(r):
        for j in range(8):
            buf[0, r, pl.ds(j * L, L)] = jnp.zeros((L,), jnp.float32)

    rows_z = n_pad // NS
    for z in range(rows_z // CHD):
        pltpu.sync_copy(buf.at[0], acc.at[pl.ds(s * rows_z + z * CHD, CHD)])
    plsc.subcore_barrier()

    def fetch_idx(k, h):
        sl = pl.ds((base_chunk + k) * CHD, CHD)
        pltpu.async_copy(row_hbm.at[sl], ibuf.at[h, 0], gsem.at[h])
        pltpu.async_copy(col_hbm.at[sl], ibuf.at[h, 1], gsem.at[h])
        pltpu.async_copy(w_hbm.at[sl], ibuf.at[h, 2], gsem.at[h])

    def wait_idx(k, h):
        sl = pl.ds((base_chunk + k) * CHD, CHD)
        pltpu.make_async_copy(row_hbm.at[sl], ibuf.at[h, 0], gsem.at[h]).wait()
        pltpu.make_async_copy(col_hbm.at[sl], ibuf.at[h, 1], gsem.at[h]).wait()
        pltpu.make_async_copy(w_hbm.at[sl], ibuf.at[h, 2], gsem.at[h]).wait()

    def scale(h):
        @pl.loop(0, CHD)
        def _(e):
            wi = plsc.load_gather(ibuf.at[h, 2],
                                  [jnp.zeros((L,), jnp.int32) + e])
            wsp = plsc.bitcast(wi, jnp.float32)
            for j in range(8):
                sl = (h, e, pl.ds(j * L, L))
                buf[sl] = buf[sl] * wsp

    fetch_idx(0, 0)
    fetch_idx(1, 1)
    wait_idx(0, 0)
    pltpu.async_copy(g_hbm.at[ibuf.at[0, 0]], buf.at[0], gsem.at[0])

    @pl.loop(0, per_tile, step=NBUF)
    def _(k0):
        for h in range(NBUF):
            k = k0 + h
            hn = (h + 1) % NBUF
            hg = (h + 2) % NBUF

            @pl.when(k >= 2)
            def _():
                pltpu.make_async_copy(
                    buf.at[hg], acc.at[ibuf.at[hg, 1]], ssem.at[hg]).wait()

            @pl.when(k + 2 < per_tile)
            def _():
                fetch_idx(k + 2, hg)

            @pl.when(k + 1 < per_tile)
            def _():
                wait_idx(k + 1, hn)
                pltpu.async_copy(g_hbm.at[ibuf.at[hn, 0]], buf.at[hn],
                                 gsem.at[hn])

            pltpu.make_async_copy(g_hbm.at[ibuf.at[h, 0]], buf.at[h],
                                  gsem.at[h]).wait()
            scale(h)
            pltpu.async_copy(buf.at[h], acc.at[ibuf.at[h, 1]], ssem.at[h],
                             add=True)

    for h in (NBUF - 2, NBUF - 1):
        pltpu.make_async_copy(
            buf.at[h], acc.at[ibuf.at[h, 1]], ssem.at[h]).wait()

    plsc.subcore_barrier()
    pltpu.sync_copy(acc.at[pl.ds(s * rows_z, rows_z)],
                    out_hbm.at[c, pl.ds(s * rows_z, rows_z)])


def kernel(x, edge_index, edge_weight, time_diff, is_weekend,
           workday_freq, weekday_freq, W_gcn, b_gcn,
           bn_gamma, bn_beta, W_lin, b_lin):
    n, c_in = x.shape
    c_out = W_gcn.shape[1]
    t_feat = W_gcn.shape[0] - c_in
    e = edge_index.shape[1]

    n_pad = ((n + NW * L - 1) // (NW * L)) * (NW * L)
    align = 81920
    e_pad = ((e + align - 1) // align) * align
    n_chunks = e_pad // CHUNK
    pad = e_pad - e
    pad_idx = jnp.arange(pad, dtype=jnp.int32) % n
    row_p = jnp.concatenate([edge_index[0], pad_idx])
    col_p = jnp.concatenate([edge_index[1], pad_idx])
    w_p = jnp.concatenate([edge_weight, jnp.zeros((pad,), jnp.float32)])
    col2d = col_p.reshape(n_chunks, CHUNK)
    w2d = w_p.reshape(n_chunks, CHUNK)

    enc = jnp.where(
        jnp.asarray(is_weekend),
        jnp.sin(time_diff * workday_freq * jnp.pi),
        jnp.cos(time_diff * weekday_freq * jnp.pi),
    )
    enc_col = enc.reshape(t_feat, 1)
    W1 = W_gcn[:c_in]
    W2 = W_gcn[c_in:]

    mesh = plsc.VectorSubcoreMesh(core_axis_name="core",
                                  subcore_axis_name="subcore",
                                  num_cores=NC, num_subcores=NS)
    sc_params = pltpu.CompilerParams()
    if "needs_layout_passes" in pltpu.CompilerParams.__dataclass_fields__:
        sc_params = dataclasses.replace(sc_params, needs_layout_passes=False)
    per_tile_b = n_chunks // NS
    dinv_pad = pl.kernel(
        functools.partial(_deg_dinv_body, n_pad, n_chunks),
        out_type=jax.ShapeDtypeStruct((n_pad,), jnp.float32),
        mesh=mesh,
        scratch_types=[
            pltpu.VMEM((per_tile_b, CHUNK), jnp.int32),
            pltpu.VMEM((per_tile_b, CHUNK), jnp.float32),
            pltpu.VMEM((n_pad // NS,), jnp.float32),
            pltpu.VMEM_SHARED((n_pad,), jnp.float32),
            pltpu.SemaphoreType.DMA,
        ],
        compiler_params=sc_params,
    )(col2d, w2d)
    dinv_col = dinv_pad[:n].reshape(n, 1)

    g = pl.pallas_call(
        _a_body,
        out_shape=jax.ShapeDtypeStruct((n, c_out), jnp.float32),
    )(x, W1, enc_col, W2, dinv_col)

    nchd = e_pad // CHD
    w_bits = lax.bitcast_convert_type(w_p, jnp.int32)
    partials = pl.kernel(
        functools.partial(_agg_body, n_pad, nchd),
        out_type=jax.ShapeDtypeStruct((NC, n_pad, c_out), jnp.float32),
        mesh=mesh,
        scratch_types=[
            pltpu.VMEM((NBUF, 3, CHD), jnp.int32),
            pltpu.VMEM((NBUF, CHD, c_out), jnp.float32),
            pltpu.VMEM_SHARED((n_pad, c_out), jnp.float32),
            pltpu.SemaphoreType.DMA((NBUF,)),
            pltpu.SemaphoreType.DMA((NBUF,)),
        ],
        compiler_params=sc_params,
    )(g, row_p, col_p, w_bits)

    out = pl.pallas_call(
        _e_body,
        out_shape=jax.ShapeDtypeStruct((n, c_out), jnp.float32),
    )(partials, g, dinv_col, b_gcn.reshape(1, c_out),
      bn_gamma.reshape(1, c_out), bn_beta.reshape(1, c_out),
      W_lin, b_lin.reshape(1, c_out))
    return out


# speedup vs baseline: 1.6138x; 1.0040x over previous
import dataclasses
import functools

import jax
import jax.numpy as jnp
from jax import lax
from jax.experimental import pallas as pl
from jax.experimental.pallas import tpu as pltpu
from jax.experimental.pallas import tpu_sc as plsc

NC, NS, L = 2, 16, 16
NW = NC * NS
CHUNK = 128


def _a_body(x_ref, w1_ref, enc_ref, w2_ref, degp_ref, g_ref, dinv_ref):
    c = jnp.sum(enc_ref[...] * w2_ref[...], axis=0, keepdims=True)
    ht = (
        jnp.dot(x_ref[...], w1_ref[...], preferred_element_type=jnp.float32)
        + c
    )
    n = ht.shape[0]
    deg = degp_ref[0, :n, :] + degp_ref[1, :n, :] + 1.0
    dinv = lax.rsqrt(deg)
    dinv_ref[...] = dinv
    g_ref[...] = ht * dinv


def _e_body(p_ref, g_ref, dinv_ref, bg_ref, gam_ref, bet_ref, wl_ref,
            bl_ref, out_ref):
    n = out_ref.shape[0]
    agg = p_ref[0, :n, :] + p_ref[1, :n, :] + g_ref[...]
    out0 = agg * dinv_ref[...] + bg_ref[...]
    mean = jnp.mean(out0, axis=0, keepdims=True)
    d = out0 - mean
    var = jnp.mean(d * d, axis=0, keepdims=True)
    y = d * lax.rsqrt(var + 1e-5) * gam_ref[...] + bet_ref[...]
    y = jnp.where(y >= 0, y, 0.01 * y)
    out_ref[...] = (
        jnp.dot(y, wl_ref[...], preferred_element_type=jnp.float32)
        + bl_ref[...]
    )


def _deg_body(n_pad, n_chunks, col_hbm, w_hbm, degp_hbm,
              colv, wv, dbuf, acc, sem):
    c = lax.axis_index("core")
    s = lax.axis_index("subcore")
    wid = c * NS + s
    per_tile = n_chunks // NW
    base_chunk = wid * per_tile
    rows_z = n_pad // NS
    pltpu.async_copy(col_hbm.at[pl.ds(base_chunk, per_tile)], colv, sem).wait()
    pltpu.async_copy(w_hbm.at[pl.ds(base_chunk, per_tile)], wv, sem).wait()

    @pl.loop(0, rows_z, step=L)
    def _(i):
        dbuf[pl.ds(i, L)] = jnp.zeros((L,), jnp.float32)

    pltpu.sync_copy(dbuf, acc.at[pl.ds(s * rows_z, rows_z)])
    plsc.subcore_barrier()

    @pl.loop(0, per_tile)
    def _(k):
        pltpu.async_copy(wv.at[k], acc.at[colv.at[k]], sem, add=True)

    @pl.loop(0, per_tile)
    def _(k):
        pltpu.make_async_copy(wv.at[k], acc.at[colv.at[k]], sem).wait()

    plsc.subcore_barrier()
    pltpu.sync_copy(acc.at[pl.ds(s * rows_z, rows_z)],
                    degp_hbm.at[c, pl.ds(s * rows_z, rows_z)])


NBUF = 4
CHD = 80


def _agg_body(n_pad, n_chunks, g_hbm, row_hbm, col_hbm, w_hbm, out_hbm,
              ibuf, buf, acc, gsem, ssem):
    c = lax.axis_index("core")
    s = lax.axis_index("subcore")
    wid = c * NS + s
    per_tile = n_chunks // NW
    base_chunk = wid * per_tile

    @pl.loop(0, CHD)
    def _(r):
        for j in range(8):
            buf[0, r, pl.ds(j * L, L)] = jnp.zeros((L,), jnp.float32)

    rows_z = n_pad // NS
    for z in range(rows_z // CHD):
        pltpu.sync_copy(buf.at[0], acc.at[pl.ds(s * rows_z + z * CHD, CHD)])
    plsc.subcore_barrier()

    def fetch_idx(k, h):
        sl = pl.ds((base_chunk + k) * CHD, CHD)
        pltpu.async_copy(row_hbm.at[sl], ibuf.at[h, 0], gsem.at[h])
        pltpu.async_copy(col_hbm.at[sl], ibuf.at[h, 1], gsem.at[h])
        pltpu.async_copy(w_hbm.at[sl], ibuf.at[h, 2], gsem.at[h])

    def wait_idx(k, h):
        sl = pl.ds((base_chunk + k) * CHD, CHD)
        pltpu.make_async_copy(row_hbm.at[sl], ibuf.at[h, 0], gsem.at[h]).wait()
        pltpu.make_async_copy(col_hbm.at[sl], ibuf.at[h, 1], gsem.at[h]).wait()
        pltpu.make_async_copy(w_hbm.at[sl], ibuf.at[h, 2], gsem.at[h]).wait()

    def scale(h):
        @pl.loop(0, CHD)
        def _(e):
            wi = plsc.load_gather(ibuf.at[h, 2],
                                  [jnp.zeros((L,), jnp.int32) + e])
            wsp = plsc.bitcast(wi, jnp.float32)
            for j in range(8):
                sl = (h, e, pl.ds(j * L, L))
                buf[sl] = buf[sl] * wsp

    fetch_idx(0, 0)
    fetch_idx(1, 1)
    wait_idx(0, 0)
    pltpu.async_copy(g_hbm.at[ibuf.at[0, 0]], buf.at[0], gsem.at[0])

    @pl.loop(0, per_tile, step=NBUF)
    def _(k0):
        for h in range(NBUF):
            k = k0 + h
            hn = (h + 1) % NBUF
            hg = (h + 2) % NBUF

            @pl.when(k >= 2)
            def _():
                pltpu.make_async_copy(
                    buf.at[hg], acc.at[ibuf.at[hg, 1]], ssem.at[hg]).wait()

            @pl.when(k + 2 < per_tile)
            def _():
                fetch_idx(k + 2, hg)

            @pl.when(k + 1 < per_tile)
            def _():
                wait_idx(k + 1, hn)
                pltpu.async_copy(g_hbm.at[ibuf.at[hn, 0]], buf.at[hn],
                                 gsem.at[hn])

            pltpu.make_async_copy(g_hbm.at[ibuf.at[h, 0]], buf.at[h],
                                  gsem.at[h]).wait()
            scale(h)
            pltpu.async_copy(buf.at[h], acc.at[ibuf.at[h, 1]], ssem.at[h],
                             add=True)

    for h in (NBUF - 2, NBUF - 1):
        pltpu.make_async_copy(
            buf.at[h], acc.at[ibuf.at[h, 1]], ssem.at[h]).wait()

    plsc.subcore_barrier()
    pltpu.sync_copy(acc.at[pl.ds(s * rows_z, rows_z)],
                    out_hbm.at[c, pl.ds(s * rows_z, rows_z)])


def kernel(x, edge_index, edge_weight, time_diff, is_weekend,
           workday_freq, weekday_freq, W_gcn, b_gcn,
           bn_gamma, bn_beta, W_lin, b_lin):
    n, c_in = x.shape
    c_out = W_gcn.shape[1]
    t_feat = W_gcn.shape[0] - c_in
    e = edge_index.shape[1]

    n_pad = ((n + NW * L - 1) // (NW * L)) * (NW * L)
    align = 81920
    e_pad = ((e + align - 1) // align) * align
    n_chunks = e_pad // CHUNK
    pad = e_pad - e
    pad_idx = jnp.arange(pad, dtype=jnp.int32) % n
    row_p = jnp.concatenate([edge_index[0], pad_idx])
    col_p = jnp.concatenate([edge_index[1], pad_idx])
    w_p = jnp.concatenate([edge_weight, jnp.zeros((pad,), jnp.float32)])
    col2d = col_p.reshape(n_chunks, CHUNK)
    w2d = w_p.reshape(n_chunks, CHUNK)

    enc = jnp.where(
        jnp.asarray(is_weekend),
        jnp.sin(time_diff * workday_freq * jnp.pi),
        jnp.cos(time_diff * weekday_freq * jnp.pi),
    )
    enc_col = enc.reshape(t_feat, 1)
    W1 = W_gcn[:c_in]
    W2 = W_gcn[c_in:]

    mesh = plsc.VectorSubcoreMesh(core_axis_name="core",
                                  subcore_axis_name="subcore",
                                  num_cores=NC, num_subcores=NS)
    sc_params = pltpu.CompilerParams()
    if "needs_layout_passes" in pltpu.CompilerParams.__dataclass_fields__:
        sc_params = dataclasses.replace(sc_params, needs_layout_passes=False)
    per_tile_b = n_chunks // NW
    degp = pl.kernel(
        functools.partial(_deg_body, n_pad, n_chunks),
        out_type=jax.ShapeDtypeStruct((NC, n_pad), jnp.float32),
        mesh=mesh,
        scratch_types=[
            pltpu.VMEM((per_tile_b, CHUNK), jnp.int32),
            pltpu.VMEM((per_tile_b, CHUNK), jnp.float32),
            pltpu.VMEM((n_pad // NS,), jnp.float32),
            pltpu.VMEM_SHARED((n_pad,), jnp.float32),
            pltpu.SemaphoreType.DMA,
        ],
        compiler_params=sc_params,
    )(col2d, w2d)

    g, dinv_col = pl.pallas_call(
        _a_body,
        out_shape=(jax.ShapeDtypeStruct((n, c_out), jnp.float32),
                   jax.ShapeDtypeStruct((n, 1), jnp.float32)),
    )(x, W1, enc_col, W2, degp.reshape(NC, n_pad, 1))

    nchd = e_pad // CHD
    w_bits = lax.bitcast_convert_type(w_p, jnp.int32)
    partials = pl.kernel(
        functools.partial(_agg_body, n_pad, nchd),
        out_type=jax.ShapeDtypeStruct((NC, n_pad, c_out), jnp.float32),
        mesh=mesh,
        scratch_types=[
            pltpu.VMEM((NBUF, 3, CHD), jnp.int32),
            pltpu.VMEM((NBUF, CHD, c_out), jnp.float32),
            pltpu.VMEM_SHARED((n_pad, c_out), jnp.float32),
            pltpu.SemaphoreType.DMA((NBUF,)),
            pltpu.SemaphoreType.DMA((NBUF,)),
        ],
        compiler_params=sc_params,
    )(g, row_p, col_p, w_bits)

    out = pl.pallas_call(
        _e_body,
        out_shape=jax.ShapeDtypeStruct((n, c_out), jnp.float32),
    )(partials, g, dinv_col, b_gcn.reshape(1, c_out),
      bn_gamma.reshape(1, c_out), bn_beta.reshape(1, c_out),
      W_lin, b_lin.reshape(1, c_out))
    return out
